# baseline (speedup 1.0000x reference)
import functools

import jax
import jax.numpy as jnp
from jax import lax
from jax.experimental import pallas as pl
from jax.experimental.pallas import tpu as pltpu
from jax.experimental.pallas import tpu_sc as plsc

N = 320000
D = 128
L = 16
NC = 2
NS = 16
NW = NC * NS
ROWS_PER_W = N // NW
CHUNK = 200
NCHUNK = ROWS_PER_W // CHUNK

_mesh = plsc.VectorSubcoreMesh(core_axis_name="c", subcore_axis_name="s")


@functools.partial(
    pl.kernel,
    mesh=_mesh,
    out_type=jax.ShapeDtypeStruct((NW * D,), jnp.float32),
    scratch_types=[
        pltpu.VMEM((CHUNK, D), jnp.float32),
        pltpu.VMEM((D,), jnp.float32),
    ],
)
def _partial_sums(data_hbm, out_hbm, buf, accv):
    wid = lax.axis_index("s") * NC + lax.axis_index("c")
    base = wid * ROWS_PER_W

    def chunk_body(ci, accs):
        start = pl.multiple_of(base + ci * CHUNK, 8)
        pltpu.sync_copy(data_hbm.at[pl.ds(start, CHUNK)], buf)

        def row_body(r, a):
            return tuple(a[j] + buf[r, pl.ds(j * L, L)] for j in range(D // L))

        return lax.fori_loop(0, CHUNK, row_body, accs)

    zero = jnp.zeros((L,), jnp.float32)
    accs = lax.fori_loop(0, NCHUNK, chunk_body, (zero,) * (D // L))
    for j in range(D // L):
        accv[pl.ds(j * L, L)] = accs[j]
    pltpu.sync_copy(accv, out_hbm.at[pl.ds(pl.multiple_of(wid * D, 8), D)])


@functools.partial(
    pl.kernel,
    mesh=_mesh,
    out_type=jax.ShapeDtypeStruct((1, D), jnp.float32),
    scratch_types=[
        pltpu.VMEM((NW * D,), jnp.float32),
        pltpu.VMEM((1, D), jnp.float32),
    ],
)
def _finalize(part_hbm, out_hbm, buf, outv):
    wid = lax.axis_index("s") * NC + lax.axis_index("c")

    @pl.when(wid == 0)
    def _():
        pltpu.sync_copy(part_hbm, buf)
        inv_n = jnp.float32(1.0 / N)
        for j in range(D // L):
            def row_body(r, a):
                return a + buf[pl.ds(r * D + j * L, L)]

            s = lax.fori_loop(0, NW, row_body, jnp.zeros((L,), jnp.float32))
            outv[0, pl.ds(j * L, L)] = s * inv_n
        pltpu.sync_copy(outv, out_hbm)


def kernel(data):
    return _finalize(_partial_sums(data))


# speedup vs baseline: 1.6181x; 1.6181x over previous
import functools

import jax
import jax.numpy as jnp
from jax import lax
from jax.experimental import pallas as pl
from jax.experimental.pallas import tpu as pltpu
from jax.experimental.pallas import tpu_sc as plsc

N = 320000
D = 128
L = 16
NC = 2
NS = 16
NW = NC * NS
ROWS_PER_W = N // NW
CHUNK = 200
NCHUNK = ROWS_PER_W // CHUNK

_mesh = plsc.VectorSubcoreMesh(core_axis_name="c", subcore_axis_name="s")


@functools.partial(
    pl.kernel,
    mesh=_mesh,
    out_type=jax.ShapeDtypeStruct((NW * D,), jnp.float32),
    scratch_types=[
        pltpu.VMEM((2, CHUNK, D), jnp.float32),
        pltpu.VMEM((D,), jnp.float32),
        pltpu.SemaphoreType.DMA,
        pltpu.SemaphoreType.DMA,
    ],
)
def _partial_sums(data_hbm, out_hbm, buf, accv, sem0, sem1):
    wid = lax.axis_index("s") * NC + lax.axis_index("c")
    base = wid * ROWS_PER_W
    sems = (sem0, sem1)
    UR = 4

    def issue(ci, b):
        start = pl.multiple_of(base + ci * CHUNK, 8)
        pltpu.async_copy(data_hbm.at[pl.ds(start, CHUNK)], buf.at[b], sems[b])

    issue(0, 0)
    issue(1, 1)

    def pair_body(pi, accs):
        for b in range(2):
            ci = pi * 2 + b
            pltpu.make_async_copy(
                data_hbm.at[pl.ds(0, CHUNK)], buf.at[b], sems[b]
            ).wait()

            def row_body(r, a):
                for u in range(UR):
                    a = tuple(
                        a[j] + buf[b, r * UR + u, pl.ds(j * L, L)]
                        for j in range(D // L)
                    )
                return a

            accs = lax.fori_loop(0, CHUNK // UR, row_body, accs)

            @pl.when(ci + 2 < NCHUNK)
            def _():
                issue(ci + 2, b)
        return accs

    zero = jnp.zeros((L,), jnp.float32)
    accs = lax.fori_loop(0, NCHUNK // 2, pair_body, (zero,) * (D // L))
    for j in range(D // L):
        accv[pl.ds(j * L, L)] = accs[j]
    pltpu.sync_copy(accv, out_hbm.at[pl.ds(pl.multiple_of(wid * D, 8), D)])


@functools.partial(
    pl.kernel,
    mesh=_mesh,
    out_type=jax.ShapeDtypeStruct((1, D), jnp.float32),
    scratch_types=[
        pltpu.VMEM((NW * D,), jnp.float32),
        pltpu.VMEM((1, D), jnp.float32),
    ],
)
def _finalize(part_hbm, out_hbm, buf, outv):
    wid = lax.axis_index("s") * NC + lax.axis_index("c")

    @pl.when(wid == 0)
    def _():
        pltpu.sync_copy(part_hbm, buf)
        inv_n = jnp.float32(1.0 / N)
        for j in range(D // L):
            def row_body(r, a):
                return a + buf[pl.ds(r * D + j * L, L)]

            s = lax.fori_loop(0, NW, row_body, jnp.zeros((L,), jnp.float32))
            outv[0, pl.ds(j * L, L)] = s * inv_n
        pltpu.sync_copy(outv, out_hbm)


def kernel(data):
    return _finalize(_partial_sums(data))


# speedup vs baseline: 1.6201x; 1.0013x over previous
import functools

import jax
import jax.numpy as jnp
from jax import lax
from jax.experimental import pallas as pl
from jax.experimental.pallas import tpu as pltpu
from jax.experimental.pallas import tpu_sc as plsc

N = 320000
D = 128
L = 16
NC = 2
NS = 16
NW = NC * NS
ROWS_PER_W = N // NW
CHUNK = 200
NCHUNK = ROWS_PER_W // CHUNK

_mesh = plsc.VectorSubcoreMesh(core_axis_name="c", subcore_axis_name="s")


@functools.partial(
    pl.kernel,
    mesh=_mesh,
    out_type=jax.ShapeDtypeStruct((NW * D,), jnp.float32),
    scratch_types=[
        pltpu.VMEM((2, CHUNK, D), jnp.float32),
        pltpu.VMEM((D,), jnp.float32),
        pltpu.SemaphoreType.DMA,
        pltpu.SemaphoreType.DMA,
    ],
)
def _partial_sums(data_hbm, out_hbm, buf, accv, sem0, sem1):
    wid = lax.axis_index("s") * NC + lax.axis_index("c")
    base = wid * ROWS_PER_W
    sems = (sem0, sem1)
    UR = 8

    def issue(ci, b):
        start = pl.multiple_of(base + ci * CHUNK, 8)
        pltpu.async_copy(data_hbm.at[pl.ds(start, CHUNK)], buf.at[b], sems[b])

    issue(0, 0)
    issue(1, 1)

    def pair_body(pi, accs):
        for b in range(2):
            ci = pi * 2 + b
            pltpu.make_async_copy(
                data_hbm.at[pl.ds(0, CHUNK)], buf.at[b], sems[b]
            ).wait()

            def row_body(r, a):
                for u in range(UR):
                    a = tuple(
                        a[j] + buf[b, r * UR + u, pl.ds(j * L, L)]
                        for j in range(D // L)
                    )
                return a

            accs = lax.fori_loop(0, CHUNK // UR, row_body, accs)

            @pl.when(ci + 2 < NCHUNK)
            def _():
                issue(ci + 2, b)
        return accs

    zero = jnp.zeros((L,), jnp.float32)
    accs = lax.fori_loop(0, NCHUNK // 2, pair_body, (zero,) * (D // L))
    for j in range(D // L):
        accv[pl.ds(j * L, L)] = accs[j]
    pltpu.sync_copy(accv, out_hbm.at[pl.ds(pl.multiple_of(wid * D, 8), D)])


@functools.partial(
    pl.kernel,
    mesh=_mesh,
    out_type=jax.ShapeDtypeStruct((1, D), jnp.float32),
    scratch_types=[
        pltpu.VMEM((NW * D,), jnp.float32),
        pltpu.VMEM((1, D), jnp.float32),
    ],
)
def _finalize(part_hbm, out_hbm, buf, outv):
    wid = lax.axis_index("s") * NC + lax.axis_index("c")

    @pl.when(wid == 0)
    def _():
        pltpu.sync_copy(part_hbm, buf)
        inv_n = jnp.float32(1.0 / N)
        for j in range(D // L):
            def row_body(r, a):
                return a + buf[pl.ds(r * D + j * L, L)]

            s = lax.fori_loop(0, NW, row_body, jnp.zeros((L,), jnp.float32))
            outv[0, pl.ds(j * L, L)] = s * inv_n
        pltpu.sync_copy(outv, out_hbm)


def kernel(data):
    return _finalize(_partial_sums(data))
